# Initial kernel scaffold; baseline (speedup 1.0000x reference)
import jax
import jax.numpy as jnp
from jax.experimental import pallas as pl


def kernel(x, edge_attr, edge_index, batch, global_init, node_enc_W1, node_enc_b1, node_enc_W2, node_enc_b2, edge_enc_W1, edge_enc_b1, edge_enc_W2, edge_enc_b2, edge_W1, edge_b1, edge_W2, edge_b2, node_W1, node_b1, node_W2, node_b2, glob_W1, glob_b1, glob_W2, glob_b2, pos_W1, pos_b1, pos_W2, pos_b2):
    raise NotImplementedError("write your pallas kernel here")


# speedup vs baseline: 4.1537x; 4.1537x over previous
import functools

import jax
import jax.numpy as jnp
from jax import lax
from jax.experimental import pallas as pl
from jax.experimental.pallas import tpu as pltpu
from jax.experimental.pallas import tpu_sc as plsc

N = 10000
E = 160000
G = 128
L = 128

BN = 2000
BE = 1600
NB_N = N // BN
NB_E = E // BE

NC = 2
NS = 16
NW = NC * NS
CH = 128
NCHUNK = E // CH
CZ = 80
NZCHUNK = N // CZ

_f32 = jnp.float32


def _mlp2_body(x_ref, w1_ref, b1_ref, w2_ref, b2_ref, o_ref):
    h = jnp.dot(x_ref[...], w1_ref[...], preferred_element_type=_f32)
    h = jnp.maximum(h + b1_ref[...], 0.0)
    h = jnp.dot(h, w2_ref[...], preferred_element_type=_f32) + b2_ref[...]
    o_ref[...] = jnp.maximum(h, 0.0)


def _encoder(x, w1, b1, w2, b2, rows, block):
    nb = rows // block
    k = x.shape[1]
    return pl.pallas_call(
        _mlp2_body,
        grid=(nb,),
        in_specs=[
            pl.BlockSpec((block, k), lambda i: (i, 0)),
            pl.BlockSpec(w1.shape, lambda i: (0, 0)),
            pl.BlockSpec(b1.shape, lambda i: (0, 0)),
            pl.BlockSpec(w2.shape, lambda i: (0, 0)),
            pl.BlockSpec(b2.shape, lambda i: (0, 0)),
        ],
        out_specs=pl.BlockSpec((block, L), lambda i: (i, 0)),
        out_shape=jax.ShapeDtypeStruct((rows, L), _f32),
    )(x, w1, b1, w2, b2)


def _nodeprep_body(h_ref, b3_ref, u_ref, wa_ref, wb_ref, wd_ref, b1_ref,
                   ha_ref, hb_ref):
    b = b3_ref[...].reshape(BN, 1)
    oh = (b == lax.broadcasted_iota(jnp.int32, (BN, G), 1)).astype(_f32)
    ud = jnp.dot(u_ref[...], wd_ref[...], preferred_element_type=_f32)
    ha = jnp.dot(h_ref[...], wa_ref[...], preferred_element_type=_f32)
    ha = ha + jnp.dot(oh, ud, preferred_element_type=_f32) + b1_ref[...]
    ha_ref[...] = ha
    hb_ref[...] = jnp.dot(h_ref[...], wb_ref[...], preferred_element_type=_f32)


def _nodeprep(h, batch3, u, wa, wb, wd, b1):
    return pl.pallas_call(
        _nodeprep_body,
        grid=(NB_N,),
        in_specs=[
            pl.BlockSpec((BN, L), lambda i: (i, 0)),
            pl.BlockSpec((1, 1, BN), lambda i: (i, 0, 0)),
            pl.BlockSpec((G, L), lambda i: (0, 0)),
            pl.BlockSpec((L, L), lambda i: (0, 0)),
            pl.BlockSpec((L, L), lambda i: (0, 0)),
            pl.BlockSpec((L, L), lambda i: (0, 0)),
            pl.BlockSpec((1, L), lambda i: (0, 0)),
        ],
        out_specs=[
            pl.BlockSpec((BN, L), lambda i: (i, 0)),
            pl.BlockSpec((BN, L), lambda i: (i, 0)),
        ],
        out_shape=[
            jax.ShapeDtypeStruct((N, L), _f32),
            jax.ShapeDtypeStruct((N, L), _f32),
        ],
    )(h, batch3, u, wa, wb, wd, b1)


def _edge_body(ta_ref, tb_ref, e_ref, wc_ref, w2_ref, b2_ref, o_ref):
    pre = ta_ref[...] + tb_ref[...] + jnp.dot(
        e_ref[...], wc_ref[...], preferred_element_type=_f32)
    mid = jnp.maximum(pre, 0.0)
    out = jnp.dot(mid, w2_ref[...], preferred_element_type=_f32) + b2_ref[...]
    o_ref[...] = e_ref[...] + jnp.maximum(out, 0.0)


def _edge_update(ta, tb, e, wc, w2, b2):
    return pl.pallas_call(
        _edge_body,
        grid=(NB_E,),
        in_specs=[
            pl.BlockSpec((BE, L), lambda i: (i, 0)),
            pl.BlockSpec((BE, L), lambda i: (i, 0)),
            pl.BlockSpec((BE, L), lambda i: (i, 0)),
            pl.BlockSpec((L, L), lambda i: (0, 0)),
            pl.BlockSpec((L, L), lambda i: (0, 0)),
            pl.BlockSpec((1, L), lambda i: (0, 0)),
        ],
        out_specs=pl.BlockSpec((BE, L), lambda i: (i, 0)),
        out_shape=jax.ShapeDtypeStruct((E, L), _f32),
    )(ta, tb, e, wc, w2, b2)


def _node_body(h_ref, recv_ref, sent_ref, b3_ref, u_ref, pos_ref,
               na_ref, nb_ref, nc_ref, nd_ref, nb1_ref, nw2_ref, nb2_ref,
               pw1_ref, pb1_ref, pw2_ref, pb2_ref,
               hn_ref, pn_ref, nagg_ref, eagg_ref):
    i = pl.program_id(0)
    b = b3_ref[...].reshape(BN, 1)
    oh = (b == lax.broadcasted_iota(jnp.int32, (BN, G), 1)).astype(_f32)
    ud = jnp.dot(u_ref[...], nd_ref[...], preferred_element_type=_f32)
    hid = (jnp.dot(h_ref[...], na_ref[...], preferred_element_type=_f32)
           + jnp.dot(recv_ref[...], nb_ref[...], preferred_element_type=_f32)
           + jnp.dot(sent_ref[...], nc_ref[...], preferred_element_type=_f32)
           + jnp.dot(oh, ud, preferred_element_type=_f32) + nb1_ref[...])
    hid = jnp.maximum(hid, 0.0)
    dn = jnp.dot(hid, nw2_ref[...], preferred_element_type=_f32) + nb2_ref[...]
    h_new = h_ref[...] + jnp.maximum(dn, 0.0)
    hn_ref[...] = h_new
    pmid = jnp.maximum(
        jnp.dot(h_new, pw1_ref[...], preferred_element_type=_f32) + pb1_ref[...],
        0.0)
    pn_ref[...] = pos_ref[...] + jnp.dot(
        pmid, pw2_ref[...], preferred_element_type=_f32) + pb2_ref[...]

    @pl.when(i == 0)
    def _():
        nagg_ref[...] = jnp.zeros_like(nagg_ref)
        eagg_ref[...] = jnp.zeros_like(eagg_ref)

    ohT_h = jax.lax.dot_general(oh, h_new, (((0,), (0,)), ((), ())),
                                preferred_element_type=_f32)
    ohT_s = jax.lax.dot_general(oh, sent_ref[...], (((0,), (0,)), ((), ())),
                                preferred_element_type=_f32)
    nagg_ref[...] += ohT_h
    eagg_ref[...] += ohT_s


def _node_update(h, recv, sent, batch3, u, pos,
                 na, nb, nc, nd, nb1, nw2, nb2, pw1, pb1, pw2, pb2):
    full = lambda a: pl.BlockSpec(a.shape, lambda i: tuple(0 for _ in a.shape))
    return pl.pallas_call(
        _node_body,
        grid=(NB_N,),
        in_specs=[
            pl.BlockSpec((BN, L), lambda i: (i, 0)),
            pl.BlockSpec((BN, L), lambda i: (i, 0)),
            pl.BlockSpec((BN, L), lambda i: (i, 0)),
            pl.BlockSpec((1, 1, BN), lambda i: (i, 0, 0)),
            pl.BlockSpec((G, L), lambda i: (0, 0)),
            pl.BlockSpec((BN, 3), lambda i: (i, 0)),
            full(na), full(nb), full(nc), full(nd), full(nb1),
            full(nw2), full(nb2), full(pw1), full(pb1), full(pw2), full(pb2),
        ],
        out_specs=[
            pl.BlockSpec((BN, L), lambda i: (i, 0)),
            pl.BlockSpec((BN, 3), lambda i: (i, 0)),
            pl.BlockSpec((G, L), lambda i: (0, 0)),
            pl.BlockSpec((G, L), lambda i: (0, 0)),
        ],
        out_shape=[
            jax.ShapeDtypeStruct((N, L), _f32),
            jax.ShapeDtypeStruct((N, 3), _f32),
            jax.ShapeDtypeStruct((G, L), _f32),
            jax.ShapeDtypeStruct((G, L), _f32),
        ],
    )(h, recv, sent, batch3, u, pos,
      na, nb, nc, nd, nb1, nw2, nb2, pw1, pb1, pw2, pb2)


def _glob_body(u_ref, nagg_ref, eagg_ref, ga_ref, gb_ref, gc_ref,
               gb1_ref, gw2_ref, gb2_ref, o_ref):
    hid = (jnp.dot(u_ref[...], ga_ref[...], preferred_element_type=_f32)
           + jnp.dot(nagg_ref[...], gb_ref[...], preferred_element_type=_f32)
           + jnp.dot(eagg_ref[...], gc_ref[...], preferred_element_type=_f32)
           + gb1_ref[...])
    hid = jnp.maximum(hid, 0.0)
    du = jnp.dot(hid, gw2_ref[...], preferred_element_type=_f32) + gb2_ref[...]
    o_ref[...] = u_ref[...] + jnp.maximum(du, 0.0)


def _glob_update(u, nagg, eagg, ga, gb, gc, gb1, gw2, gb2):
    full = lambda a: pl.BlockSpec(a.shape, lambda: tuple(0 for _ in a.shape))
    return pl.pallas_call(
        _glob_body,
        in_specs=[full(u), full(nagg), full(eagg), full(ga), full(gb),
                  full(gc), full(gb1), full(gw2), full(gb2)],
        out_specs=full(u),
        out_shape=jax.ShapeDtypeStruct((G, L), _f32),
    )(u, nagg, eagg, ga, gb, gc, gb1, gw2, gb2)


_MESH = plsc.VectorSubcoreMesh(core_axis_name="c", subcore_axis_name="s")


def _gather_body(ha_hbm, hb_hbm, src_hbm, dst_hbm, oa_hbm, ob_hbm,
                 idxa, idxb, bufa, bufb, sema, semb):
    cid = lax.axis_index("c")
    sid = lax.axis_index("s")
    wid = sid * NC + cid
    nfull = NCHUNK // NW
    rem = NCHUNK - nfull * NW
    n_w = nfull + jnp.where(wid < rem, 1, 0)

    def body(k, carry):
        j = wid + k * NW
        base = j * CH
        pltpu.sync_copy(src_hbm.at[pl.ds(base, CH)], idxa)
        pltpu.sync_copy(dst_hbm.at[pl.ds(base, CH)], idxb)
        ca = pltpu.async_copy(ha_hbm.at[idxa], bufa, sema)
        cb = pltpu.async_copy(hb_hbm.at[idxb], bufb, semb)
        ca.wait()
        pltpu.sync_copy(bufa, oa_hbm.at[pl.ds(base, CH)])
        cb.wait()
        pltpu.sync_copy(bufb, ob_hbm.at[pl.ds(base, CH)])
        return carry

    lax.fori_loop(0, n_w, body, 0)


@functools.partial(
    pl.kernel,
    mesh=_MESH,
    out_type=[
        jax.ShapeDtypeStruct((E, L), _f32),
        jax.ShapeDtypeStruct((E, L), _f32),
    ],
    scratch_types=[
        pltpu.VMEM((CH,), jnp.int32),
        pltpu.VMEM((CH,), jnp.int32),
        pltpu.VMEM((CH, L), _f32),
        pltpu.VMEM((CH, L), _f32),
        pltpu.SemaphoreType.DMA,
        pltpu.SemaphoreType.DMA,
    ],
)
def _sc_gather(ha_hbm, hb_hbm, src_hbm, dst_hbm, oa_hbm, ob_hbm,
               idxa, idxb, bufa, bufb, sema, semb):
    _gather_body(ha_hbm, hb_hbm, src_hbm, dst_hbm, oa_hbm, ob_hbm,
                 idxa, idxb, bufa, bufb, sema, semb)


def _scatter_one(e_hbm, idx_hbm, out_hbm, idx, buf, zbuf, table, sid):
    z = jnp.zeros((16,), _f32)

    def zb(i, carry):
        r = i // 8
        c = (i % 8) * 16
        zbuf[r, pl.ds(c, 16)] = z
        return carry

    lax.fori_loop(0, CZ * 8, zb, 0)

    nz_full = NZCHUNK // NS
    nz_rem = NZCHUNK - nz_full * NS
    nz_w = nz_full + jnp.where(sid < nz_rem, 1, 0)

    def zt(k, carry):
        j = sid + k * NS
        pltpu.sync_copy(zbuf, table.at[pl.ds(j * CZ, CZ)])
        return carry

    lax.fori_loop(0, nz_w, zt, 0)
    plsc.subcore_barrier()

    nfull = NCHUNK // NS
    rem = NCHUNK - nfull * NS
    n_w = nfull + jnp.where(sid < rem, 1, 0)

    def body(k, carry):
        j = sid + k * NS
        base = j * CH
        pltpu.sync_copy(idx_hbm.at[pl.ds(base, CH)], idx)
        pltpu.sync_copy(e_hbm.at[pl.ds(base, CH)], buf)
        pltpu.sync_copy(buf, table.at[idx], add=True)
        return carry

    lax.fori_loop(0, n_w, body, 0)
    plsc.subcore_barrier()

    def out(k, carry):
        j = sid + k * NS
        pltpu.sync_copy(table.at[pl.ds(j * CZ, CZ)], out_hbm.at[pl.ds(j * CZ, CZ)])
        return carry

    lax.fori_loop(0, nz_w, out, 0)


@functools.partial(
    pl.kernel,
    mesh=_MESH,
    out_type=[
        jax.ShapeDtypeStruct((N, L), _f32),
        jax.ShapeDtypeStruct((N, L), _f32),
    ],
    scratch_types=[
        pltpu.VMEM((CH,), jnp.int32),
        pltpu.VMEM((CH, L), _f32),
        pltpu.VMEM((CZ, L), _f32),
        pltpu.VMEM_SHARED((N, L), _f32),
    ],
)
def _sc_scatter(e_hbm, src_hbm, dst_hbm, recv_hbm, sent_hbm,
                idx, buf, zbuf, table):
    cid = lax.axis_index("c")
    sid = lax.axis_index("s")

    @pl.when(cid == 0)
    def _():
        _scatter_one(e_hbm, dst_hbm, recv_hbm, idx, buf, zbuf, table, sid)

    @pl.when(cid == 1)
    def _():
        _scatter_one(e_hbm, src_hbm, sent_hbm, idx, buf, zbuf, table, sid)


def kernel(x, edge_attr, edge_index, batch, global_init,
           node_enc_W1, node_enc_b1, node_enc_W2, node_enc_b2,
           edge_enc_W1, edge_enc_b1, edge_enc_W2, edge_enc_b2,
           edge_W1, edge_b1, edge_W2, edge_b2,
           node_W1, node_b1, node_W2, node_b2,
           glob_W1, glob_b1, glob_W2, glob_b2,
           pos_W1, pos_b1, pos_W2, pos_b2):
    S = edge_W1.shape[0]
    src = edge_index[0]
    dst = edge_index[1]
    batch3 = batch.reshape(NB_N, 1, BN)
    r1 = lambda b: b.reshape(1, -1)

    h = _encoder(x, node_enc_W1, r1(node_enc_b1), node_enc_W2,
                 r1(node_enc_b2), N, BN)
    e = _encoder(edge_attr, edge_enc_W1, r1(edge_enc_b1), edge_enc_W2,
                 r1(edge_enc_b2), E, BE)
    u = jnp.tile(global_init, (G, 1))
    pos = jnp.zeros((N, 3), _f32)

    for i in range(S):
        eWa, eWb, eWc, eWd = (edge_W1[i, 0:L], edge_W1[i, L:2 * L],
                              edge_W1[i, 2 * L:3 * L], edge_W1[i, 3 * L:4 * L])
        ha, hb = _nodeprep(h, batch3, u, eWa, eWb, eWd, r1(edge_b1[i]))
        ta, tb = _sc_gather(ha, hb, src, dst)
        e = _edge_update(ta, tb, e, eWc, edge_W2[i], r1(edge_b2[i]))
        recv, sent = _sc_scatter(e, src, dst)
        nA, nB, nC, nD = (node_W1[i, 0:L], node_W1[i, L:2 * L],
                          node_W1[i, 2 * L:3 * L], node_W1[i, 3 * L:4 * L])
        h, pos, nagg, eagg = _node_update(
            h, recv, sent, batch3, u, pos,
            nA, nB, nC, nD, r1(node_b1[i]), node_W2[i], r1(node_b2[i]),
            pos_W1[i], r1(pos_b1[i]), pos_W2[i], r1(pos_b2[i]))
        gA, gB, gC = (glob_W1[i, 0:L], glob_W1[i, L:2 * L],
                      glob_W1[i, 2 * L:3 * L])
        u = _glob_update(u, nagg, eagg, gA, gB, gC, r1(glob_b1[i]),
                         glob_W2[i], r1(glob_b2[i]))
    return pos
